# Initial kernel scaffold; baseline (speedup 1.0000x reference)
import jax
import jax.numpy as jnp
from jax.experimental import pallas as pl


def kernel(features, neigh_idx, W, b):
    raise NotImplementedError("write your pallas kernel here")


# speedup vs baseline: 1.2081x; 1.2081x over previous
import functools

import jax
import jax.numpy as jnp
from jax import lax
from jax.experimental import pallas as pl
from jax.experimental.pallas import tpu as pltpu
from jax.experimental.pallas import tpu_sc as plsc

N = 50000
D = 128
K = 9
OUT = 128

NC = 2
NS = 16
NW = NC * NS

BN = 512
NPAD = 50176
RPW = NPAD // NW
CH = 16
NCH = RPW // CH
IDXC = CH * K
HALF = IDXC // 2
GROUP = 14
GROWS = GROUP * CH


def _matmul_body(f_ref, wt_ref, z_ref):
    z_ref[0] = jnp.dot(f_ref[...], wt_ref[0], preferred_element_type=jnp.float32)


def _tc_matmul(features_pad, wt):
    return pl.pallas_call(
        _matmul_body,
        grid=(NPAD // BN, K),
        in_specs=[
            pl.BlockSpec((BN, D), lambda i, k: (i, 0)),
            pl.BlockSpec((1, D, OUT), lambda i, k: (k, 0, 0)),
        ],
        out_specs=pl.BlockSpec((1, BN, OUT), lambda i, k: (k, i, 0)),
        out_shape=jax.ShapeDtypeStruct((K, NPAD, OUT), jnp.float32),
    )(features_pad, wt)


def _sc_body(z_hbm, gidx_hbm, b_hbm, out_hbm, idx_v, g_v, og_v, bias_v, sem0, sem1):
    cid = lax.axis_index("c")
    sid = lax.axis_index("s")
    w = cid * NS + sid
    base_row = w * RPW

    pltpu.sync_copy(gidx_hbm.at[pl.ds(base_row * K, RPW * K)], idx_v)
    pltpu.sync_copy(b_hbm, bias_v)
    bias_vecs = [bias_v[pl.ds(c * 16, 16)] for c in range(OUT // 16)]
    sems = (sem0, sem1)

    def issue(chunk, buf):
        off = chunk * IDXC
        for h in range(2):
            pltpu.async_copy(
                z_hbm.at[idx_v.at[pl.ds(off + h * HALF, HALF)]],
                g_v.at[buf, pl.ds(h * HALF, HALF)],
                sems[buf],
            )

    def wait_gather(buf):
        pltpu.make_async_copy(
            z_hbm.at[pl.ds(0, IDXC)], g_v.at[buf], sems[buf]
        ).wait()

    issue(0, 0)
    issue(1, 1)

    def outer(t, carry):
        for buf in range(2):
            chunk = t * 2 + buf
            wait_gather(buf)

            def row_body(r, c2):
                gbase = r * K
                orow = (chunk % GROUP) * CH + r
                for c in range(OUT // 16):
                    lanes = pl.ds(c * 16, 16)
                    acc = g_v[buf, gbase, lanes] + bias_vecs[c]
                    for k in range(1, K):
                        acc = acc + g_v[buf, gbase + k, lanes]
                    og_v[orow, lanes] = acc
                return c2

            lax.fori_loop(0, CH, row_body, 0)

            @pl.when(chunk + 2 < NCH)
            def _():
                issue(chunk + 2, buf)

            @pl.when(chunk % GROUP == GROUP - 1)
            def _():
                grp = chunk // GROUP
                pltpu.sync_copy(
                    og_v, out_hbm.at[pl.ds(base_row + grp * GROWS, GROWS)]
                )
        return carry

    lax.fori_loop(0, NCH // 2, outer, 0)


def _sc_gather_accum(z_flat, gidx, b):
    mesh = plsc.VectorSubcoreMesh(
        core_axis_name="c", subcore_axis_name="s", num_cores=NC, num_subcores=NS
    )
    kern = functools.partial(
        pl.kernel,
        out_type=jax.ShapeDtypeStruct((NPAD, OUT), jnp.float32),
        mesh=mesh,
        scratch_types=[
            pltpu.VMEM((RPW * K,), jnp.int32),
            pltpu.VMEM((2, IDXC, OUT), jnp.float32),
            pltpu.VMEM((GROWS, OUT), jnp.float32),
            pltpu.VMEM((OUT,), jnp.float32),
            pltpu.SemaphoreType.DMA,
            pltpu.SemaphoreType.DMA,
        ],
    )(_sc_body)
    return kern(z_flat, gidx, b)


def kernel(features, neigh_idx, W, b):
    features_pad = jnp.zeros((NPAD, D), jnp.float32).at[:N].set(features)
    wt = W.reshape(OUT, K, D).transpose(1, 2, 0)

    idx_safe = jnp.where(neigh_idx < 0, N, neigh_idx).astype(jnp.int32)
    gidx = idx_safe + (jnp.arange(K, dtype=jnp.int32) * NPAD)[None, :]
    gidx = jnp.full((NPAD, K), N, jnp.int32).at[:N].set(gidx).reshape(-1)

    z = _tc_matmul(features_pad, wt)
    z_flat = z.reshape(K * NPAD, OUT)
    out_pad = _sc_gather_accum(z_flat, gidx, b)
    return out_pad[:N]
